# baseline (speedup 1.0000x reference)
import functools

import jax
import jax.numpy as jnp
from jax import lax
from jax.experimental import pallas as pl
from jax.experimental.pallas import tpu as pltpu
from jax.experimental.pallas import tpu_sc as plsc

NUM_CORES = 2
NUM_SUBCORES = 16
NW = NUM_CORES * NUM_SUBCORES
LANES = 16

BATCH = 1024
SEQ = 200
D = 64
TOTAL = BATCH * SEQ
RPW = TOTAL // NW
W = 128
CPW = RPW // W
NGROUPS = D // LANES


@functools.partial(jax.jit, static_argnames=())
def _sc_embed(token_table, idx3, pos2):
    mesh = plsc.VectorSubcoreMesh(core_axis_name="c", subcore_axis_name="s")

    @functools.partial(
        pl.kernel,
        out_type=jax.ShapeDtypeStruct((TOTAL, D), jnp.float32),
        mesh=mesh,
        scratch_types=[
            pltpu.VMEM((CPW, W), jnp.int32),
            pltpu.VMEM((2 * SEQ, D), jnp.float32),
            pltpu.VMEM((W, D), jnp.float32),
            pltpu.VMEM((W, D), jnp.float32),
            pltpu.SemaphoreType.DMA,
            pltpu.SemaphoreType.DMA,
        ],
        compiler_params=pltpu.CompilerParams(use_tc_tiling_on_sc=False),
    )
    def k(table_hbm, idx_hbm, pos_hbm, out_hbm,
          idx_v, pos_v, buf0, buf1, sem0, sem1):
        wid = lax.axis_index("s") * NUM_CORES + lax.axis_index("c")
        base = wid * RPW
        pltpu.sync_copy(idx_hbm.at[wid], idx_v)
        pltpu.sync_copy(pos_hbm, pos_v)

        bufs = (buf0, buf1)
        sems = (sem0, sem1)

        pltpu.async_copy(table_hbm.at[idx_v.at[0]], buf0, sem0)

        @pl.loop(0, CPW, step=2)
        def _(c):
            for b in range(2):
                buf, sem = bufs[b], sems[b]
                cc = c + b
                pltpu.make_async_copy(
                    table_hbm.at[idx_v.at[cc]], buf, sem).wait()
                nxt = cc + 1

                @pl.when(nxt < CPW)
                def _():
                    pltpu.async_copy(
                        table_hbm.at[idx_v.at[nxt]], bufs[1 - b],
                        sems[1 - b])

                poff = lax.rem(cc * W, SEQ)

                @pl.loop(0, W)
                def _(r):
                    for g in range(NGROUPS):
                        sl = pl.ds(g * LANES, LANES)
                        buf[r, sl] = buf[r, sl] + pos_v[poff + r, sl]

                pltpu.sync_copy(buf, out_hbm.at[pl.ds(base + cc * W, W)])

    return k(token_table, idx3, pos2)


def kernel(inputs, token_table, pos_table):
    idx3 = inputs.reshape(NW, CPW, W).astype(jnp.int32)
    pos2 = jnp.concatenate([pos_table, pos_table], axis=0)
    out = _sc_embed(token_table, idx3, pos2)
    return out.reshape(BATCH, SEQ, D)


# speedup vs baseline: 1.1078x; 1.1078x over previous
import functools

import jax
import jax.numpy as jnp
from jax import lax
from jax.experimental import pallas as pl
from jax.experimental.pallas import tpu as pltpu
from jax.experimental.pallas import tpu_sc as plsc

NUM_CORES = 2
NUM_SUBCORES = 16
NW = NUM_CORES * NUM_SUBCORES
BATCH = 1024
SEQ = 200
D = 64
TOTAL = BATCH * SEQ
RPW = TOTAL // NW
W = 128
CPW = RPW // W
NBUF = 5
assert CPW % NBUF == 0


def _sc_gather(token_table, idx3):
    mesh = plsc.VectorSubcoreMesh(core_axis_name="c", subcore_axis_name="s")

    @functools.partial(
        pl.kernel,
        out_type=jax.ShapeDtypeStruct((TOTAL, D), jnp.float32),
        mesh=mesh,
        scratch_types=(
            [pltpu.VMEM((CPW, W), jnp.int32)]
            + [pltpu.VMEM((W, D), jnp.float32) for _ in range(NBUF)]
            + [pltpu.SemaphoreType.DMA for _ in range(2 * NBUF)]
        ),
        compiler_params=pltpu.CompilerParams(use_tc_tiling_on_sc=False),
    )
    def k(table_hbm, idx_hbm, out_hbm, idx_v, *bufs_and_sems):
        bufs = bufs_and_sems[:NBUF]
        gsem = bufs_and_sems[NBUF:2 * NBUF]
        wsem = bufs_and_sems[2 * NBUF:3 * NBUF]

        wid = lax.axis_index("s") * NUM_CORES + lax.axis_index("c")
        base = wid * RPW
        pltpu.sync_copy(idx_hbm.at[wid], idx_v)

        def gather(cc, b):
            pltpu.async_copy(table_hbm.at[idx_v.at[cc]], bufs[b], gsem[b])

        def wait_gather(cc, b):
            pltpu.make_async_copy(
                table_hbm.at[idx_v.at[cc]], bufs[b], gsem[b]).wait()

        def write(cc, b):
            pltpu.async_copy(
                bufs[b], out_hbm.at[pl.ds(base + cc * W, W)], wsem[b])

        def wait_write(cc, b):
            pltpu.make_async_copy(
                bufs[b], out_hbm.at[pl.ds(base + cc * W, W)], wsem[b]).wait()

        for b in range(NBUF - 1):
            gather(b, b)

        @pl.loop(0, CPW, step=NBUF)
        def _(c):
            for b in range(NBUF):
                cc = c + b
                wait_gather(cc, b)
                write(cc, b)
                bp = (b - 1) % NBUF

                @pl.when(cc >= 1)
                def _():
                    wait_write(cc - 1, bp)

                @pl.when(cc - 1 + NBUF < CPW)
                def _():
                    gather(cc - 1 + NBUF, bp)

        wait_write(CPW - 1, (CPW - 1) % NBUF)

    return k(token_table, idx3)


def kernel(inputs, token_table, pos_table):
    idx3 = inputs.reshape(NW, CPW, W).astype(jnp.int32)
    gathered = _sc_gather(token_table, idx3)
    return gathered.reshape(BATCH, SEQ, D) + pos_table[None, :, :]


# speedup vs baseline: 1.2751x; 1.1511x over previous
import functools

import jax
import jax.numpy as jnp
from jax import lax
from jax.experimental import pallas as pl
from jax.experimental.pallas import tpu as pltpu
from jax.experimental.pallas import tpu_sc as plsc

NUM_CORES = 2
NUM_SUBCORES = 16
NW = NUM_CORES * NUM_SUBCORES
BATCH = 1024
SEQ = 200
D = 64
TOTAL = BATCH * SEQ
RPW = TOTAL // NW
W = 128
CPW = RPW // W
NBUF = 5
assert CPW % NBUF == 0


def _sc_gather(token_table, idx3):
    mesh = plsc.VectorSubcoreMesh(core_axis_name="c", subcore_axis_name="s")

    @functools.partial(
        pl.kernel,
        out_type=jax.ShapeDtypeStruct((TOTAL, 2 * D), jnp.float32),
        mesh=mesh,
        scratch_types=(
            [pltpu.VMEM((CPW, W), jnp.int32)]
            + [pltpu.VMEM((W, 2 * D), jnp.float32) for _ in range(NBUF)]
            + [pltpu.SemaphoreType.DMA for _ in range(2 * NBUF)]
        ),
        compiler_params=pltpu.CompilerParams(use_tc_tiling_on_sc=True),
    )
    def k(table_hbm, idx_hbm, out_hbm, idx_v, *bufs_and_sems):
        bufs = bufs_and_sems[:NBUF]
        gsem = bufs_and_sems[NBUF:2 * NBUF]
        wsem = bufs_and_sems[2 * NBUF:3 * NBUF]

        wid = lax.axis_index("s") * NUM_CORES + lax.axis_index("c")
        base = wid * RPW
        pltpu.sync_copy(idx_hbm.at[wid], idx_v)

        def gather(cc, b):
            pltpu.async_copy(table_hbm.at[idx_v.at[cc]], bufs[b], gsem[b])

        def wait_gather(cc, b):
            pltpu.make_async_copy(
                table_hbm.at[idx_v.at[cc]], bufs[b], gsem[b]).wait()

        def write(cc, b):
            pltpu.async_copy(
                bufs[b], out_hbm.at[pl.ds(base + cc * W, W)], wsem[b])

        def wait_write(cc, b):
            pltpu.make_async_copy(
                bufs[b], out_hbm.at[pl.ds(base + cc * W, W)], wsem[b]).wait()

        for b in range(NBUF - 1):
            gather(b, b)

        @pl.loop(0, CPW, step=NBUF)
        def _(c):
            for b in range(NBUF):
                cc = c + b
                wait_gather(cc, b)
                write(cc, b)
                bp = (b - 1) % NBUF

                @pl.when(cc >= 1)
                def _():
                    wait_write(cc - 1, bp)

                @pl.when(cc - 1 + NBUF < CPW)
                def _():
                    gather(cc - 1 + NBUF, bp)

        wait_write(CPW - 1, (CPW - 1) % NBUF)

    return k(token_table, idx3)


def kernel(inputs, token_table, pos_table):
    idx3 = inputs.reshape(NW, CPW, W).astype(jnp.int32)
    tpad = jnp.pad(token_table, ((0, 0), (0, D)))
    gathered = _sc_gather(tpad, idx3)
    gathered = gathered.reshape(BATCH, SEQ, 2 * D)[:, :, :D]
    return gathered + pos_table[None, :, :]
